# Initial kernel scaffold; baseline (speedup 1.0000x reference)
import jax
import jax.numpy as jnp
from jax.experimental import pallas as pl


def kernel(x, edge_index, batch, poi_embed_table, fc_w, fc_b, hidden_adj, hidden_feat, mlp_w, mlp_b):
    raise NotImplementedError("write your pallas kernel here")


# speedup vs baseline: 5.6145x; 5.6145x over previous
import functools

import jax
import jax.numpy as jnp
import numpy as np
from jax import lax
from jax.experimental import pallas as pl
from jax.experimental.pallas import tpu as pltpu
from jax.experimental.pallas import tpu_sc as plsc

MAX_STEP = 3
HID_DIM = 128
HGN = 16
HGS = 10
N_NODES = 10000
N_EDGES = 320000
N_GRAPH_IDS = 128

D = HID_DIM
K = HGN * HGS
NG = N_GRAPH_IDS

NC = 2
NS = 16
NW = NC * NS
CH = 128

NP = 10240
BLK = 512
NBLK = NP // BLK
NCHUNK_G = NP // CH

EPT = N_EDGES // NW
CPT = 80
EPT_P = CPT * CH
ROWS_PER_TILE = NP // NS

@functools.cache
def _mesh():
    return plsc.VectorSubcoreMesh(
        core_axis_name="c", subcore_axis_name="s",
        num_cores=NC, num_subcores=NS)


@functools.cache
def _sc_gather_fn():
    return pl.kernel(
        _sc_gather_body,
        out_type=jax.ShapeDtypeStruct((NP, D), jnp.float32),
        mesh=_mesh(),
        scratch_types=[
            pltpu.VMEM((CH,), jnp.int32),
            pltpu.VMEM((CH, D), jnp.float32),
            pltpu.SemaphoreType.DMA,
        ],
    )


def _sc_gather_body(idx_hbm, table_hbm, out_hbm, idx_v, rows_v, sem):
    w = lax.axis_index("c") * NS + lax.axis_index("s")
    for j in range(-(-NCHUNK_G // NW)):
        chunk = w + j * NW

        @pl.when(chunk < NCHUNK_G)
        def _():
            pltpu.sync_copy(idx_hbm.at[pl.ds(chunk * CH, CH)], idx_v)
            pltpu.async_copy(table_hbm.at[idx_v], rows_v, sem).wait()
            pltpu.sync_copy(rows_v, out_hbm.at[pl.ds(chunk * CH, CH)])


@functools.cache
def _sc_scatter_fn():
    return pl.kernel(
        _sc_scatter_body,
        out_type=jax.ShapeDtypeStruct((NC, NP, D), jnp.float32),
        mesh=_mesh(),
        scratch_types=[
            pltpu.VMEM((CPT, CH), jnp.int32),
            pltpu.VMEM((CPT, CH), jnp.int32),
            pltpu.VMEM((CH, D), jnp.float32),
            pltpu.VMEM_SHARED((NP, D), jnp.float32),
            pltpu.SemaphoreType.DMA,
        ],
    )


def _sc_scatter_body(src_hbm, dst_hbm, xx_hbm, zeros_hbm, out_hbm,
                     src_v, dst_v, rows_v, acc, sem):
    c = lax.axis_index("c")
    s = lax.axis_index("s")
    w = c * NS + s
    pltpu.sync_copy(src_hbm.at[pl.ds(w * CPT, CPT)], src_v)
    pltpu.sync_copy(dst_hbm.at[pl.ds(w * CPT, CPT)], dst_v)
    pltpu.sync_copy(zeros_hbm, acc.at[pl.ds(s * ROWS_PER_TILE, ROWS_PER_TILE)])
    plsc.subcore_barrier()

    def body(j, carry):
        pltpu.async_copy(xx_hbm.at[src_v.at[j]], rows_v, sem).wait()
        pltpu.sync_copy(rows_v, acc.at[dst_v.at[j]], add=True)
        return carry

    lax.fori_loop(0, CPT, body, 0)
    plsc.subcore_barrier()
    pltpu.sync_copy(acc.at[pl.ds(s * ROWS_PER_TILE, ROWS_PER_TILE)],
                    out_hbm.at[c, pl.ds(s * ROWS_PER_TILE, ROWS_PER_TILE)])


def _dot(a, b):
    return jax.lax.dot_general(
        a, b, (((1,), (0,)), ((), ())),
        precision=jax.lax.Precision.HIGHEST,
        preferred_element_type=jnp.float32)


def _tc_a_body(pf, fcwT, fcb, z0rT, xx_out, g0_out):
    v = _dot(pf[...], fcwT[...]) + fcb[...]
    xx = jax.nn.sigmoid(v)
    xx_out[...] = xx
    g0_out[...] = _dot(xx, z0rT[...])


def _tc_a(pf, fcwT, fcb, z0rT):
    return pl.pallas_call(
        _tc_a_body,
        grid=(NBLK,),
        in_specs=[
            pl.BlockSpec((BLK, D), lambda i: (i, 0)),
            pl.BlockSpec((D, D), lambda i: (0, 0)),
            pl.BlockSpec((1, D), lambda i: (0, 0)),
            pl.BlockSpec((D, K), lambda i: (0, 0)),
        ],
        out_specs=[
            pl.BlockSpec((BLK, D), lambda i: (i, 0)),
            pl.BlockSpec((BLK, K), lambda i: (i, 0)),
        ],
        out_shape=[
            jax.ShapeDtypeStruct((NP, D), jnp.float32),
            jax.ShapeDtypeStruct((NP, K), jnp.float32),
        ],
    )(pf, fcwT, fcb, z0rT)


def _tc_b_body(p, z1rT, xx_out, g1_out):
    xx = p[0] + p[1]
    xx_out[...] = xx
    g1_out[...] = _dot(xx, z1rT[...])


def _tc_b(p, z1rT):
    return pl.pallas_call(
        _tc_b_body,
        grid=(NBLK,),
        in_specs=[
            pl.BlockSpec((NC, BLK, D), lambda i: (0, i, 0)),
            pl.BlockSpec((D, K), lambda i: (0, 0)),
        ],
        out_specs=[
            pl.BlockSpec((BLK, D), lambda i: (i, 0)),
            pl.BlockSpec((BLK, K), lambda i: (i, 0)),
        ],
        out_shape=[
            jax.ShapeDtypeStruct((NP, D), jnp.float32),
            jax.ShapeDtypeStruct((NP, K), jnp.float32),
        ],
    )(p, z1rT)


def _leaky(v):
    return jnp.where(v >= 0, v, 0.01 * v)


def _tc_final_body(p2, z2rT, g0, g1, batch3, mlpT, mlpb, out,
                   seg0, seg1, seg2):
    i = pl.program_id(0)
    g2 = _dot(p2[0] + p2[1], z2rT[...])
    g0v = g0[...]
    g1v = g1[...]
    bb = batch3[0]
    oh = (lax.broadcasted_iota(jnp.int32, (NG, BLK), 0) == bb).astype(jnp.float32)

    @pl.when(i == 0)
    def _():
        seg0[...] = jnp.zeros((NG, K), jnp.float32)
        seg1[...] = jnp.zeros((NG, K), jnp.float32)
        seg2[...] = jnp.zeros((NG, K), jnp.float32)

    seg0[...] += _dot(oh, g0v * g0v)
    seg1[...] += _dot(oh, g0v * g1v)
    seg2[...] += _dot(oh, g0v * g2)

    @pl.when(i == NBLK - 1)
    def _():
        sel = (lax.broadcasted_iota(jnp.int32, (K, HGN), 0) // HGS
               == lax.broadcasted_iota(jnp.int32, (K, HGN), 1)).astype(jnp.float32)
        u0 = _dot(seg0[...], sel)
        u1 = _dot(seg1[...], sel)
        u2 = _dot(seg2[...], sel)
        v = (_dot(u0, mlpT[0:HGN, :]) + _dot(u1, mlpT[HGN:2 * HGN, :])
             + _dot(u2, mlpT[2 * HGN:3 * HGN, :]) + mlpb[...])
        out[...] = _leaky(v)


def _tc_final(p2, z2rT, g0, g1, batch3, mlpT, mlpb):
    return pl.pallas_call(
        _tc_final_body,
        grid=(NBLK,),
        in_specs=[
            pl.BlockSpec((NC, BLK, D), lambda i: (0, i, 0)),
            pl.BlockSpec((D, K), lambda i: (0, 0)),
            pl.BlockSpec((BLK, K), lambda i: (i, 0)),
            pl.BlockSpec((BLK, K), lambda i: (i, 0)),
            pl.BlockSpec((1, 1, BLK), lambda i: (i, 0, 0)),
            pl.BlockSpec((MAX_STEP * HGN, D), lambda i: (0, 0)),
            pl.BlockSpec((1, D), lambda i: (0, 0)),
        ],
        out_specs=pl.BlockSpec((NG, D), lambda i: (0, 0)),
        out_shape=jax.ShapeDtypeStruct((NG, D), jnp.float32),
        scratch_shapes=[
            pltpu.VMEM((NG, K), jnp.float32),
            pltpu.VMEM((NG, K), jnp.float32),
            pltpu.VMEM((NG, K), jnp.float32),
        ],
    )(p2, z2rT, g0, g1, batch3, mlpT, mlpb)


def kernel(x, edge_index, batch, poi_embed_table, fc_w, fc_b,
           hidden_adj, hidden_feat, mlp_w, mlp_b):
    f32 = jnp.float32
    iu0, iu1 = np.triu_indices(HGS, 1)
    adj = jnp.zeros((HGN, HGS, HGS), f32).at[:, iu0, iu1].set(_leaky(hidden_adj))
    adj = adj + jnp.transpose(adj, (0, 2, 1))
    z0 = hidden_feat
    z1 = jnp.einsum('abc,acd->abd', adj, z0)
    z2 = jnp.einsum('abc,acd->abd', adj, z1)
    z0rT = z0.reshape(K, D).T
    z1rT = z1.reshape(K, D).T
    z2rT = z2.reshape(K, D).T
    fcwT = fc_w.T
    fcb = fc_b.reshape(1, D)
    mlpT = mlp_w.T
    mlpb = mlp_b.reshape(1, D)

    xg = jnp.pad(x.astype(jnp.int32), (0, NP - N_NODES))
    src = edge_index[0].reshape(NW, EPT)
    dst = edge_index[1].reshape(NW, EPT)
    src = jnp.pad(src, ((0, 0), (0, EPT_P - EPT))).reshape(NW * CPT, CH)
    dst = jnp.pad(dst, ((0, 0), (0, EPT_P - EPT)),
                  constant_values=N_NODES).reshape(NW * CPT, CH)
    batch3 = jnp.pad(batch, (0, NP - N_NODES),
                     constant_values=NG).reshape(NBLK, 1, BLK)
    zrows = jnp.zeros((ROWS_PER_TILE, D), f32)

    pf = _sc_gather_fn()(xg, poi_embed_table)
    xx0, g0 = _tc_a(pf, fcwT, fcb, z0rT)
    p1 = _sc_scatter_fn()(src, dst, xx0, zrows)
    xx1, g1 = _tc_b(p1, z1rT)
    p2 = _sc_scatter_fn()(src, dst, xx1, zrows)
    return _tc_final(p2, z2rT, g0, g1, batch3, mlpT, mlpb)
